# Initial kernel scaffold; baseline (speedup 1.0000x reference)
import jax
import jax.numpy as jnp
from jax.experimental import pallas as pl


def kernel(input, adj, weight, bias):
    raise NotImplementedError("write your pallas kernel here")


# speedup vs baseline: 1.0381x; 1.0381x over previous
import jax
import jax.numpy as jnp
from jax.experimental import pallas as pl
from jax.experimental.pallas import tpu as pltpu

_BR = 400


def _gc_kernel(x_ref, w_ref, b_ref, adj_ref, out_ref, support_ref):
    @pl.when(pl.program_id(0) == 0)
    def _():
        support_ref[...] = jnp.dot(
            x_ref[...], w_ref[...], preferred_element_type=jnp.float32
        )

    out_ref[...] = (
        jnp.dot(adj_ref[...], support_ref[...], preferred_element_type=jnp.float32)
        + b_ref[...]
    )


def kernel(input, adj, weight, bias):
    n, f_in = input.shape
    f_out = weight.shape[1]
    bias2d = bias.reshape(1, f_out)
    grid = (adj.shape[0] // _BR,)
    return pl.pallas_call(
        _gc_kernel,
        grid=grid,
        in_specs=[
            pl.BlockSpec((n, f_in), lambda i: (0, 0)),
            pl.BlockSpec((f_in, f_out), lambda i: (0, 0)),
            pl.BlockSpec((1, f_out), lambda i: (0, 0)),
            pl.BlockSpec((_BR, n), lambda i: (i, 0)),
        ],
        out_specs=pl.BlockSpec((_BR, f_out), lambda i: (i, 0)),
        out_shape=jax.ShapeDtypeStruct((adj.shape[0], f_out), jnp.float32),
        scratch_shapes=[pltpu.VMEM((n, f_out), jnp.float32)],
    )(input, weight, bias2d, adj)
